# Initial kernel scaffold; baseline (speedup 1.0000x reference)
import jax
import jax.numpy as jnp
from jax.experimental import pallas as pl


def kernel(input_ids, labels, negative_samples, emb_in, emb_out):
    raise NotImplementedError("write your pallas kernel here")


# speedup vs baseline: 5.5749x; 5.5749x over previous
import jax
import jax.numpy as jnp
from jax import lax
from jax.experimental import pallas as pl
from jax.experimental.pallas import tpu as pltpu
from jax.experimental.pallas import tpu_sc as plsc

DIM = 128
B = 4096
CTX = 20
CTXP = 24
NNEG = 20
NSCORE = NNEG + 1
SCW = 32

NC = 2
NS = 16
NW = NC * NS
NB = B // NW
LANES = 16
NCH = DIM // LANES


def _sc_body(ids_hbm, outids_hbm, emb_in_hbm, emb_out_hbm, scores_hbm,
             ids_v, outids_v, ctx_bufs, out_bufs, scores_v,
             sem_c0, sem_c1, sem_o0, sem_o1):
    wid = lax.axis_index("s") * NC + lax.axis_index("c")
    base = wid * NB

    pltpu.sync_copy(ids_hbm.at[pl.ds(base, NB), :], ids_v)
    pltpu.sync_copy(outids_hbm.at[pl.ds(base, NB), :], outids_v)

    ctx_sems = (sem_c0, sem_c1)
    out_sems = (sem_o0, sem_o1)

    def ctx_copy(b, slot):
        return pltpu.make_async_copy(
            emb_in_hbm.at[ids_v.at[b, pl.ds(0, CTX)]],
            ctx_bufs.at[slot], ctx_sems[slot])

    def out_copy(b, slot):
        return pltpu.make_async_copy(
            emb_out_hbm.at[outids_v.at[b]], out_bufs.at[slot], out_sems[slot])

    for s in range(2):
        ctx_copy(s, s).start()
        out_copy(s, s).start()

    lane_iota = lax.iota(jnp.int32, LANES)
    one = jnp.float32(1.0)
    zero_f = jnp.float32(0.0)
    gdn = lax.GatherDimensionNumbers(
        offset_dims=(), collapsed_slice_dims=(0,), start_index_map=(0,))
    perms = [jnp.bitwise_xor(lane_iota, sh)[:, None] for sh in (8, 4, 2, 1)]

    def lane_sum(v):
        for perm in perms:
            g = lax.gather(v, perm, dimension_numbers=gdn, slice_sizes=(1,),
                           mode=lax.GatherScatterMode.PROMISE_IN_BOUNDS)
            v = v + g
        return v

    def compute(b, slot):
        ids_a = ids_v[b, pl.ds(0, LANES)]
        ids_b = ids_v[b, pl.ds(8, LANES)]
        m_a = jnp.where(ids_a != 0, one, zero_f)
        m_b = jnp.where(ids_b != 0, one, zero_f)

        acc = [jnp.zeros((LANES,), jnp.float32)] * NCH
        for c in range(CTX):
            m = m_a[c] if c < LANES else m_b[c - 8]
            for j in range(NCH):
                acc[j] = acc[j] + ctx_bufs[slot, c, pl.ds(j * LANES, LANES)] * m

        vec_a = jnp.zeros((LANES,), jnp.float32)
        vec_b = jnp.zeros((LANES,), jnp.float32)
        for k in range(NSCORE):
            s = out_bufs[slot, k, pl.ds(0, LANES)] * acc[0]
            for j in range(1, NCH):
                s = s + out_bufs[slot, k, pl.ds(j * LANES, LANES)] * acc[j]
            val = lane_sum(s)
            if k < LANES:
                vec_a = jnp.where(lane_iota == k, val, vec_a)
            else:
                vec_b = jnp.where(lane_iota == (k - LANES), val, vec_b)

        scores_v[b, pl.ds(0, LANES)] = vec_a
        scores_v[b, pl.ds(LANES, LANES)] = vec_b

    def step(i, carry):
        for s in range(2):
            b = i * 2 + s
            ctx_copy(b, s).wait()
            out_copy(b, s).wait()

            @pl.when(b + 2 < NB)
            def _():
                ctx_copy(b + 2, s).start()
                out_copy(b + 2, s).start()

            compute(b, s)
        return carry

    lax.fori_loop(0, NB // 2, step, 0)

    pltpu.sync_copy(scores_v, scores_hbm.at[pl.ds(base, NB), :])


@jax.jit
def _run(ids, outids, emb_in, emb_out):
    mesh = plsc.VectorSubcoreMesh(
        core_axis_name="c", subcore_axis_name="s",
        num_cores=NC, num_subcores=NS)
    kern = pl.kernel(
        _sc_body,
        out_type=jax.ShapeDtypeStruct((B, SCW), jnp.float32),
        mesh=mesh,
        scratch_types=[
            pltpu.VMEM((NB, CTXP), jnp.int32),
            pltpu.VMEM((NB, NSCORE), jnp.int32),
            pltpu.VMEM((2, CTX, DIM), jnp.float32),
            pltpu.VMEM((2, NSCORE, DIM), jnp.float32),
            pltpu.VMEM((NB, SCW), jnp.float32),
            pltpu.SemaphoreType.DMA,
            pltpu.SemaphoreType.DMA,
            pltpu.SemaphoreType.DMA,
            pltpu.SemaphoreType.DMA,
        ],
    )
    return kern(ids, outids, emb_in, emb_out)


def kernel(input_ids, labels, negative_samples, emb_in, emb_out):
    ids = jnp.pad(input_ids.astype(jnp.int32), ((0, 0), (0, CTXP - CTX)))
    outids = jnp.concatenate(
        [labels.astype(jnp.int32), negative_samples.astype(jnp.int32)], axis=1)
    scores = _run(ids, outids, emb_in, emb_out)
    return scores[:, 0], scores[:, 1:NSCORE]


# speedup vs baseline: 6.9410x; 1.2450x over previous
import jax
import jax.numpy as jnp
from jax import lax
from jax.experimental import pallas as pl
from jax.experimental.pallas import tpu as pltpu
from jax.experimental.pallas import tpu_sc as plsc

DIM = 128
B = 4096
CTX = 20
NNEG = 20

NC = 2
NS = 16
NW = NC * NS
NB = B // NW
CB = 8
NCHUNK = NB // CB
CROWS = CB * CTX
LANES = 16
NCH = DIM // LANES


def _sc_body(ids_hbm, negs_hbm, labs_hbm, emb_in_hbm, emb_out_hbm,
             pos_hbm, neg_hbm,
             ids_v, negs_v, labs_v, ctx_bufs, negrow_bufs, lab_bufs,
             pos_stage, neg_stage,
             sem_cx0, sem_cx1, sem_ng0, sem_ng1, sem_lb0, sem_lb1):
    wid = lax.axis_index("s") * NC + lax.axis_index("c")
    base = wid * NB

    pltpu.sync_copy(ids_hbm.at[pl.ds(base * CTX, NB * CTX)], ids_v)
    pltpu.sync_copy(negs_hbm.at[pl.ds(base * NNEG, NB * NNEG)], negs_v)
    pltpu.sync_copy(labs_hbm.at[pl.ds(base, NB)], labs_v)

    cx_sems = (sem_cx0, sem_cx1)
    ng_sems = (sem_ng0, sem_ng1)
    lb_sems = (sem_lb0, sem_lb1)

    def chunk_copies(k, slot):
        off = pl.multiple_of(k * CROWS, CROWS)
        half = CROWS // 2
        cps = []
        for h in range(2):
            cps.append(pltpu.make_async_copy(
                emb_in_hbm.at[ids_v.at[pl.ds(off + h * half, half)]],
                ctx_bufs.at[slot, pl.ds(h * half, half)], cx_sems[slot]))
            cps.append(pltpu.make_async_copy(
                emb_out_hbm.at[negs_v.at[pl.ds(off + h * half, half)]],
                negrow_bufs.at[slot, pl.ds(h * half, half)], ng_sems[slot]))
        cps.append(pltpu.make_async_copy(
            emb_out_hbm.at[labs_v.at[pl.ds(pl.multiple_of(k * CB, CB), CB)]],
            lab_bufs.at[slot], lb_sems[slot]))
        return cps

    for s in range(2):
        for cp in chunk_copies(s, s):
            cp.start()

    lane_iota = lax.iota(jnp.int32, LANES)
    one = jnp.float32(1.0)
    zero_f = jnp.float32(0.0)
    gdn = lax.GatherDimensionNumbers(
        offset_dims=(), collapsed_slice_dims=(0,), start_index_map=(0,))
    perms = [jnp.bitwise_xor(lane_iota, sh)[:, None] for sh in (8, 4, 2, 1)]

    def lane_sum(v):
        for perm in perms:
            g = lax.gather(v, perm, dimension_numbers=gdn, slice_sizes=(1,),
                           mode=lax.GatherScatterMode.PROMISE_IN_BOUNDS)
            v = v + g
        return v

    def dot_with(acc, row_ref, r):
        s = row_ref[r, pl.ds(0, LANES)] * acc[0]
        for j in range(1, NCH):
            s = s + row_ref[r, pl.ds(j * LANES, LANES)] * acc[j]
        return lane_sum(s)

    def chunk_body(i, s, pos_vec):
        k = i * 2 + s
        for cp in chunk_copies(k, s):
            cp.wait()

        ctx_ref = ctx_bufs.at[s]
        negrow_ref = negrow_bufs.at[s]

        def bbody(bb, pos_vec):
            b = k * CB + bb
            o = k * CROWS + bb * CTX
            ids_a = ids_v[pl.ds(o, LANES)]
            ids_b = ids_v[pl.ds(o + 4, LANES)]
            m_a = jnp.where(ids_a != 0, one, zero_f)
            m_b = jnp.where(ids_b != 0, one, zero_f)

            r0 = bb * CTX
            acc = [jnp.zeros((LANES,), jnp.float32)] * NCH
            for c in range(CTX):
                m = m_a[c] if c < LANES else m_b[c - 4]
                for j in range(NCH):
                    acc[j] = acc[j] + ctx_ref[r0 + c, pl.ds(j * LANES, LANES)] * m

            pv = dot_with(acc, lab_bufs.at[s], bb)
            pos_vec = jnp.where(lane_iota == (s * CB + bb), pv, pos_vec)

            vec_a = jnp.zeros((LANES,), jnp.float32)
            vec_c = jnp.zeros((LANES,), jnp.float32)
            for j in range(NNEG):
                bs = dot_with(acc, negrow_ref, r0 + j)
                if j < LANES:
                    vec_a = jnp.where(lane_iota == j, bs, vec_a)
                if j >= 4:
                    vec_c = jnp.where(lane_iota == (j - 4), bs, vec_c)
            neg_stage[b, pl.ds(0, LANES)] = vec_a
            neg_stage[b, pl.ds(4, LANES)] = vec_c
            return pos_vec

        pos_vec = lax.fori_loop(0, CB, bbody, pos_vec)

        @pl.when(k + 2 < NCHUNK)
        def _():
            for cp in chunk_copies(k + 2, s):
                cp.start()

        return pos_vec

    def step(i, pos_vec):
        pos_vec = chunk_body(i, 0, pos_vec)
        pos_vec = chunk_body(i, 1, pos_vec)
        pos_stage[pl.ds(i * LANES, LANES)] = pos_vec
        return jnp.zeros((LANES,), jnp.float32)

    lax.fori_loop(0, NCHUNK // 2, step, jnp.zeros((LANES,), jnp.float32))

    pltpu.sync_copy(pos_stage, pos_hbm.at[pl.ds(base, NB)])
    pltpu.sync_copy(neg_stage, neg_hbm.at[pl.ds(base, NB), :])


@jax.jit
def _run(ids, negs, labs, emb_in, emb_out):
    mesh = plsc.VectorSubcoreMesh(
        core_axis_name="c", subcore_axis_name="s",
        num_cores=NC, num_subcores=NS)
    kern = pl.kernel(
        _sc_body,
        out_type=(
            jax.ShapeDtypeStruct((B,), jnp.float32),
            jax.ShapeDtypeStruct((B, NNEG), jnp.float32),
        ),
        mesh=mesh,
        scratch_types=[
            pltpu.VMEM((NB * CTX,), jnp.int32),
            pltpu.VMEM((NB * NNEG,), jnp.int32),
            pltpu.VMEM((NB,), jnp.int32),
            pltpu.VMEM((2, CROWS, DIM), jnp.float32),
            pltpu.VMEM((2, CROWS, DIM), jnp.float32),
            pltpu.VMEM((2, CB, DIM), jnp.float32),
            pltpu.VMEM((NB,), jnp.float32),
            pltpu.VMEM((NB, NNEG), jnp.float32),
            pltpu.SemaphoreType.DMA,
            pltpu.SemaphoreType.DMA,
            pltpu.SemaphoreType.DMA,
            pltpu.SemaphoreType.DMA,
            pltpu.SemaphoreType.DMA,
            pltpu.SemaphoreType.DMA,
        ],
    )
    return kern(ids, negs, labs, emb_in, emb_out)


def kernel(input_ids, labels, negative_samples, emb_in, emb_out):
    ids = input_ids.astype(jnp.int32).reshape(B * CTX)
    negs = negative_samples.astype(jnp.int32).reshape(B * NNEG)
    labs = labels.astype(jnp.int32).reshape(B)
    return _run(ids, negs, labs, emb_in, emb_out)


# speedup vs baseline: 7.0539x; 1.0163x over previous
import jax
import jax.numpy as jnp
from jax import lax
from jax.experimental import pallas as pl
from jax.experimental.pallas import tpu as pltpu
from jax.experimental.pallas import tpu_sc as plsc

DIM = 128
B = 4096
CTX = 20
NNEG = 20

NC = 2
NS = 16
NW = NC * NS
NB = B // NW
CB = 4
NCHUNK = NB // CB
NSLOT = 4
CROWS = CB * CTX
LANES = 16
NCH = DIM // LANES

IDS_OFF = 0
NEGS_OFF = B * CTX
LABS_OFF = B * (CTX + NNEG)


def _sc_body(idx_hbm, emb_in_hbm, emb_out_hbm, pos_hbm, neg_hbm,
             ids_v, negs_v, labs_v, ctx_bufs, negrow_bufs, lab_rows,
             pos_stage, neg_stage,
             sem_lab, sem_c0, sem_c1, sem_c2, sem_c3,
             sem_n0, sem_n1, sem_n2, sem_n3):
    wid = lax.axis_index("s") * NC + lax.axis_index("c")
    base = wid * NB

    pltpu.sync_copy(idx_hbm.at[pl.ds(IDS_OFF + base * CTX, NB * CTX)], ids_v)
    pltpu.sync_copy(idx_hbm.at[pl.ds(NEGS_OFF + base * NNEG, NB * NNEG)], negs_v)
    pltpu.sync_copy(idx_hbm.at[pl.ds(LABS_OFF + base, NB)], labs_v)

    cx_sems = (sem_c0, sem_c1, sem_c2, sem_c3)
    ng_sems = (sem_n0, sem_n1, sem_n2, sem_n3)

    lab_cp = pltpu.make_async_copy(emb_out_hbm.at[labs_v], lab_rows, sem_lab)
    lab_cp.start()

    def ctx_copy(k, slot):
        off = pl.multiple_of(k * CROWS, CROWS)
        return pltpu.make_async_copy(
            emb_in_hbm.at[ids_v.at[pl.ds(off, CROWS)]],
            ctx_bufs.at[slot], cx_sems[slot])

    def neg_copy(k, slot):
        off = pl.multiple_of(k * CROWS, CROWS)
        return pltpu.make_async_copy(
            emb_out_hbm.at[negs_v.at[pl.ds(off, CROWS)]],
            negrow_bufs.at[slot], ng_sems[slot])

    for s in range(NSLOT):
        ctx_copy(s, s).start()
        neg_copy(s, s).start()
    lab_cp.wait()

    lane_iota = lax.iota(jnp.int32, LANES)
    one = jnp.float32(1.0)
    zero_f = jnp.float32(0.0)
    gdn = lax.GatherDimensionNumbers(
        offset_dims=(), collapsed_slice_dims=(0,), start_index_map=(0,))
    perms = [jnp.bitwise_xor(lane_iota, sh)[:, None] for sh in (8, 4, 2, 1)]

    def lane_sum(v):
        for perm in perms:
            g = lax.gather(v, perm, dimension_numbers=gdn, slice_sizes=(1,),
                           mode=lax.GatherScatterMode.PROMISE_IN_BOUNDS)
            v = v + g
        return v

    def dot_with(acc, row_ref, r):
        s = row_ref[r, pl.ds(0, LANES)] * acc[0]
        for j in range(1, NCH):
            s = s + row_ref[r, pl.ds(j * LANES, LANES)] * acc[j]
        return lane_sum(s)

    def chunk_body(i, s, pos_vec):
        k = i * NSLOT + s
        ctx_copy(k, s).wait()
        neg_copy(k, s).wait()

        ctx_ref = ctx_bufs.at[s]
        negrow_ref = negrow_bufs.at[s]

        def bbody(bb, pos_vec):
            b = k * CB + bb
            o = k * CROWS + bb * CTX
            ids_a = ids_v[pl.ds(o, LANES)]
            ids_b = ids_v[pl.ds(o + 4, LANES)]
            m_a = jnp.where(ids_a != 0, one, zero_f)
            m_b = jnp.where(ids_b != 0, one, zero_f)

            r0 = bb * CTX
            acc = [jnp.zeros((LANES,), jnp.float32)] * NCH
            for c in range(CTX):
                m = m_a[c] if c < LANES else m_b[c - 4]
                for j in range(NCH):
                    acc[j] = acc[j] + ctx_ref[r0 + c, pl.ds(j * LANES, LANES)] * m

            pv = dot_with(acc, lab_rows, b)
            pos_vec = jnp.where(lane_iota == (b % LANES), pv, pos_vec)

            vec_a = jnp.zeros((LANES,), jnp.float32)
            vec_c = jnp.zeros((LANES,), jnp.float32)
            for j in range(NNEG):
                bs = dot_with(acc, negrow_ref, r0 + j)
                if j < LANES:
                    vec_a = jnp.where(lane_iota == j, bs, vec_a)
                if j >= 4:
                    vec_c = jnp.where(lane_iota == (j - 4), bs, vec_c)
            neg_stage[b, pl.ds(0, LANES)] = vec_a
            neg_stage[b, pl.ds(4, LANES)] = vec_c
            return pos_vec

        pos_vec = lax.fori_loop(0, CB, bbody, pos_vec)

        @pl.when(k + NSLOT < NCHUNK)
        def _():
            ctx_copy(k + NSLOT, s).start()
            neg_copy(k + NSLOT, s).start()

        @pl.when(k % 4 == 3)
        def _():
            pos_stage[pl.ds((k // 4) * LANES, LANES)] = pos_vec

        return pos_vec

    def step(i, pos_vec):
        for s in range(NSLOT):
            pos_vec = chunk_body(i, s, pos_vec)
        return pos_vec

    lax.fori_loop(0, NCHUNK // NSLOT, step, jnp.zeros((LANES,), jnp.float32))

    pltpu.sync_copy(pos_stage, pos_hbm.at[pl.ds(base, NB)])
    pltpu.sync_copy(neg_stage, neg_hbm.at[pl.ds(base, NB), :])


@jax.jit
def _run(idx, emb_in, emb_out):
    mesh = plsc.VectorSubcoreMesh(
        core_axis_name="c", subcore_axis_name="s",
        num_cores=NC, num_subcores=NS)
    kern = pl.kernel(
        _sc_body,
        out_type=(
            jax.ShapeDtypeStruct((B,), jnp.float32),
            jax.ShapeDtypeStruct((B, NNEG), jnp.float32),
        ),
        mesh=mesh,
        scratch_types=[
            pltpu.VMEM((NB * CTX,), jnp.int32),
            pltpu.VMEM((NB * NNEG,), jnp.int32),
            pltpu.VMEM((NB,), jnp.int32),
            pltpu.VMEM((NSLOT, CROWS, DIM), jnp.float32),
            pltpu.VMEM((NSLOT, CROWS, DIM), jnp.float32),
            pltpu.VMEM((NB, DIM), jnp.float32),
            pltpu.VMEM((NB,), jnp.float32),
            pltpu.VMEM((NB, NNEG), jnp.float32),
        ] + [pltpu.SemaphoreType.DMA] * 9,
    )
    return kern(idx, emb_in, emb_out)


def kernel(input_ids, labels, negative_samples, emb_in, emb_out):
    idx = jnp.concatenate([
        input_ids.astype(jnp.int32).reshape(B * CTX),
        negative_samples.astype(jnp.int32).reshape(B * NNEG),
        labels.astype(jnp.int32).reshape(B),
    ])
    return _run(idx, emb_in, emb_out)
